# baseline (speedup 1.0000x reference)
import functools

import jax
import jax.numpy as jnp
from jax import lax
from jax.experimental import pallas as pl
from jax.experimental.pallas import tpu as pltpu
from jax.experimental.pallas import tpu_sc as plsc

_LANES = 16
_CHUNK = 16
_NSETS = 3
_PAIR = 2


def kernel(inputs, pos_table):
    batch, seq, dim = inputs.shape
    info = plsc.get_sparse_core_info()
    nw = info.num_cores * info.num_subcores
    seq_per_w = seq // nw
    n_chunks = seq_per_w // _CHUNK
    n_pairs = batch // _PAIR
    n_steps = n_chunks * n_pairs
    n_vecs = dim // _LANES
    mesh = plsc.VectorSubcoreMesh(core_axis_name="c", subcore_axis_name="s")

    scratch = (
        [pltpu.VMEM((_CHUNK, dim), jnp.float32) for _ in range(_NSETS * _PAIR)]
        + [pltpu.VMEM((_CHUNK, dim), jnp.float32)]
        + [pltpu.SemaphoreType.DMA for _ in range(_NSETS * 2 + 1)]
    )

    @functools.partial(
        pl.kernel,
        mesh=mesh,
        out_type=jax.ShapeDtypeStruct((batch, seq, dim), jnp.float32),
        scratch_types=scratch,
    )
    def sc_kernel(in_hbm, pos_hbm, out_hbm, *refs):
        bufs = [refs[r * _PAIR:(r + 1) * _PAIR] for r in range(_NSETS)]
        pos_v = refs[_NSETS * _PAIR]
        sems = refs[_NSETS * _PAIR + 1:]
        in_sems = sems[:_NSETS]
        out_sems = sems[_NSETS:2 * _NSETS]
        pos_sem = sems[2 * _NSETS]

        wid = lax.axis_index("s") * info.num_cores + lax.axis_index("c")
        seq0 = wid * seq_per_w
        steps = [(ci, pr) for ci in range(n_chunks) for pr in range(n_pairs)]

        def issue_in(s):
            ci, pr = steps[s]
            r = s % _NSETS
            row0 = seq0 + ci * _CHUNK
            return [
                pltpu.async_copy(
                    in_hbm.at[pr * _PAIR + j, pl.ds(row0, _CHUNK), :],
                    bufs[r][j], in_sems[r])
                for j in range(_PAIR)
            ]

        pos_h = pltpu.async_copy(
            pos_hbm.at[pl.ds(seq0, _CHUNK), :], pos_v, pos_sem)
        in_h = {0: issue_in(0)}
        out_h = {}
        for s, (ci, pr) in enumerate(steps):
            r = s % _NSETS
            row0 = seq0 + ci * _CHUNK
            if s >= 2:
                for h in out_h[s - 2]:
                    h.wait()
            if s + 1 < n_steps:
                in_h[s + 1] = issue_in(s + 1)
            if pr == 0:
                pos_h.wait()
            for h in in_h[s]:
                h.wait()

            bset = bufs[r]

            @plsc.parallel_loop(0, _CHUNK, 1)
            def row_body(rr, bset=bset):
                @plsc.parallel_loop(0, n_vecs, 1, unroll=4)
                def vec_body(v):
                    sl = pl.ds(v * _LANES, _LANES)
                    pv = pos_v[rr, sl]
                    for j in range(_PAIR):
                        bset[j][rr, sl] = bset[j][rr, sl] + pv

            if pr == n_pairs - 1 and ci + 1 < n_chunks:
                pos_h = pltpu.async_copy(
                    pos_hbm.at[pl.ds(seq0 + (ci + 1) * _CHUNK, _CHUNK), :],
                    pos_v, pos_sem)
            out_h[s] = [
                pltpu.async_copy(
                    bset[j], out_hbm.at[pr * _PAIR + j, pl.ds(row0, _CHUNK), :],
                    out_sems[r])
                for j in range(_PAIR)
            ]
        for s in (n_steps - 2, n_steps - 1):
            for h in out_h[s]:
                h.wait()

    return sc_kernel(inputs, pos_table)


# speedup vs baseline: 1.0563x; 1.0563x over previous
import functools

import jax
import jax.numpy as jnp
from jax import lax
from jax.experimental import pallas as pl
from jax.experimental.pallas import tpu as pltpu
from jax.experimental.pallas import tpu_sc as plsc

_LANES = 16
_CHUNK = 8
_NSETS = 3


def kernel(inputs, pos_table):
    batch, seq, dim = inputs.shape
    info = plsc.get_sparse_core_info()
    nw = info.num_cores * info.num_subcores
    seq_per_w = seq // nw
    n_chunks = seq_per_w // _CHUNK
    n_vecs = dim // _LANES
    mesh = plsc.VectorSubcoreMesh(core_axis_name="c", subcore_axis_name="s")

    scratch = (
        [pltpu.VMEM((_CHUNK, dim), jnp.float32) for _ in range(_NSETS * batch)]
        + [pltpu.VMEM((_CHUNK, dim), jnp.float32) for _ in range(2)]
        + [pltpu.SemaphoreType.DMA for _ in range(_NSETS * 2 + 2)]
    )

    @functools.partial(
        pl.kernel,
        mesh=mesh,
        out_type=jax.ShapeDtypeStruct((batch, seq, dim), jnp.float32),
        scratch_types=scratch,
    )
    def sc_kernel(in_hbm, pos_hbm, out_hbm, *refs):
        bufs = [refs[r * batch:(r + 1) * batch] for r in range(_NSETS)]
        pos_bufs = refs[_NSETS * batch:_NSETS * batch + 2]
        sems = refs[_NSETS * batch + 2:]
        in_sems = sems[:_NSETS]
        out_sems = sems[_NSETS:2 * _NSETS]
        pos_sems = sems[2 * _NSETS:]

        wid = lax.axis_index("s") * info.num_cores + lax.axis_index("c")
        seq0 = wid * seq_per_w

        def issue_in(c):
            r = c % _NSETS
            row0 = seq0 + c * _CHUNK
            hs = [
                pltpu.async_copy(
                    in_hbm.at[b, pl.ds(row0, _CHUNK), :], bufs[r][b],
                    in_sems[r])
                for b in range(batch)
            ]
            hs.append(pltpu.async_copy(
                pos_hbm.at[pl.ds(row0, _CHUNK), :], pos_bufs[c % 2],
                pos_sems[c % 2]))
            return hs

        in_h = {0: issue_in(0)}
        out_h = {}
        for c in range(n_chunks):
            r = c % _NSETS
            row0 = seq0 + c * _CHUNK
            if c >= 2:
                for h in out_h[c - 2]:
                    h.wait()
            if c + 1 < n_chunks:
                in_h[c + 1] = issue_in(c + 1)
            for h in in_h[c]:
                h.wait()

            bset = bufs[r]
            pos_b = pos_bufs[c % 2]

            @plsc.parallel_loop(0, _CHUNK, 1)
            def row_body(rr, bset=bset, pos_b=pos_b):
                @plsc.parallel_loop(0, n_vecs, 1, unroll=4)
                def vec_body(v):
                    sl = pl.ds(v * _LANES, _LANES)
                    pv = pos_b[rr, sl]
                    for b in range(batch):
                        bset[b][rr, sl] = bset[b][rr, sl] + pv

            out_h[c] = [
                pltpu.async_copy(
                    bset[b], out_hbm.at[b, pl.ds(row0, _CHUNK), :],
                    out_sems[r])
                for b in range(batch)
            ]
        for c in (n_chunks - 2, n_chunks - 1):
            for h in out_h[c]:
                h.wait()

    return sc_kernel(inputs, pos_table)
